# baseline (speedup 1.0000x reference)
import functools

import jax
import jax.numpy as jnp
from jax.experimental import pallas as pl
from jax.experimental.pallas import tpu as pltpu
from jax.experimental.pallas import tpu_sc as plsc


_CHUNK = 256
_GATHER_WINDOW = 256


def _distance_argmin(x_r, qW, qb2, emb, *, interpret=False):
    bs, c, n = x_r.shape
    k, d = emb.shape
    nchunks = k // _CHUNK

    def body(x_ref, qw_ref, qb_ref, emb_ref, idx_ref, loss_ref, epad_ref):
        b = pl.program_id(0)
        echunk = emb_ref[pl.ds(b * (k // bs), k // bs), :]
        epad_ref[...] = jnp.concatenate(
            [echunk, jnp.zeros((k // bs, 128 - d), jnp.float32)], axis=1)
        xb = x_ref[0]
        qi = jax.lax.dot_general(
            qw_ref[...].astype(jnp.bfloat16), xb.astype(jnp.bfloat16),
            (((1,), (0,)), ((), ())), preferred_element_type=jnp.float32)
        qi = qi + qb_ref[...]
        q_sq = jnp.sum(qi * qi, axis=0, keepdims=True)
        qi_b = qi.astype(jnp.bfloat16)
        run_min = jnp.full((1, n), jnp.inf, jnp.float32)
        run_idx = jnp.zeros((1, n), jnp.int32)
        rows_f = jax.lax.broadcasted_iota(
            jnp.int32, (_CHUNK, n), 0).astype(jnp.float32)
        for kk in range(nchunks):
            eb = emb_ref[pl.ds(kk * _CHUNK, _CHUNK), :]
            esq = jnp.sum(eb * eb, axis=1, keepdims=True)
            dot2 = jax.lax.dot_general(
                (eb + eb).astype(jnp.bfloat16), qi_b,
                (((1,), (0,)), ((), ())), preferred_element_type=jnp.float32)
            s = (q_sq + esq) - dot2
            cmin = jnp.min(s, axis=0, keepdims=True)
            cidx_f = jnp.min(jnp.where(s == cmin, rows_f, float(k)),
                             axis=0, keepdims=True)
            cidx = cidx_f.astype(jnp.int32) + (kk * _CHUNK)
            better = cmin < run_min
            run_idx = jnp.where(better, cidx, run_idx)
            run_min = jnp.where(better, cmin, run_min)
        idx_ref[...] = run_idx
        loss_ref[0] = jnp.sum(jnp.maximum(run_min, 0.0), axis=1, keepdims=True)

    return pl.pallas_call(
        body,
        grid=(bs,),
        in_specs=[
            pl.BlockSpec((1, c, n), lambda b: (b, 0, 0)),
            pl.BlockSpec((emb.shape[1], c), lambda b: (0, 0)),
            pl.BlockSpec((emb.shape[1], 1), lambda b: (0, 0)),
            pl.BlockSpec((k, d), lambda b: (0, 0)),
        ],
        out_specs=[
            pl.BlockSpec((1, n), lambda b: (0, b)),
            pl.BlockSpec((1, 1, 1), lambda b: (b, 0, 0)),
            pl.BlockSpec((k // bs, 128), lambda b: (b, 0)),
        ],
        out_shape=[
            jax.ShapeDtypeStruct((1, bs * n), jnp.int32),
            jax.ShapeDtypeStruct((bs, 1, 1), jnp.float32),
            jax.ShapeDtypeStruct((k, 128), jnp.float32),
        ],
        interpret=interpret,
    )(x_r, qW, qb2, emb)


def _sc_gather(emb, indices):
    total = indices.shape[0]
    k, d = emb.shape
    num_cores, num_subcores = 2, 16
    nw = num_cores * num_subcores
    b_per_w = total // nw
    mesh = plsc.VectorSubcoreMesh(core_axis_name="c", subcore_axis_name="s")

    @functools.partial(
        pl.kernel, mesh=mesh,
        out_type=jax.ShapeDtypeStruct((total, d), emb.dtype),
        scratch_types=[
            pltpu.VMEM((b_per_w,), jnp.int32),
            pltpu.VMEM((b_per_w, d), emb.dtype),
            pltpu.SemaphoreType.DMA,
        ],
    )
    def gather_kernel(emb_hbm, i_hbm, o_hbm, idx_v, rows_v, sem):
        wid = jax.lax.axis_index("s") * num_cores + jax.lax.axis_index("c")
        base = wid * b_per_w
        pltpu.sync_copy(i_hbm.at[pl.ds(base, b_per_w)], idx_v)
        pltpu.async_copy(emb_hbm.at[idx_v], rows_v, sem).wait()
        pltpu.sync_copy(rows_v, o_hbm.at[pl.ds(base, b_per_w)])

    return gather_kernel(emb, indices)


def _post_residual(x_r, q, pW, pb2, *, interpret=False):
    bs, c, n = x_r.shape
    d = pW.shape[1]
    qd = q.shape[-1]

    def body(x_ref, q_ref, pw_ref, pb_ref, out_ref):
        xb = x_ref[0]
        qv = q_ref[0]
        pw = jnp.concatenate(
            [pw_ref[...], jnp.zeros((c, qd - d), jnp.float32)], axis=1)
        post = jax.lax.dot_general(
            pw.astype(jnp.bfloat16), qv.astype(jnp.bfloat16),
            (((1,), (1,)), ((), ())), preferred_element_type=jnp.float32)
        out_ref[0] = xb - (post + pb_ref[...])

    return pl.pallas_call(
        body,
        grid=(bs,),
        in_specs=[
            pl.BlockSpec((1, c, n), lambda b: (b, 0, 0)),
            pl.BlockSpec((1, n, qd), lambda b: (b, 0, 0)),
            pl.BlockSpec((c, d), lambda b: (0, 0)),
            pl.BlockSpec((c, 1), lambda b: (0, 0)),
        ],
        out_specs=pl.BlockSpec((1, c, n), lambda b: (b, 0, 0)),
        out_shape=jax.ShapeDtypeStruct((bs, c, n), jnp.float32),
        interpret=interpret,
    )(x_r, q, pW, pb2)


def kernel(x, qW, qb, emb, pW, pb):
    bs, c, h, w = x.shape
    n = h * w
    d = emb.shape[1]
    x_r = x.reshape(bs, c, n)
    idx, loss_parts, emb_pad = _distance_argmin(x_r, qW, qb.reshape(d, 1), emb)
    quantized = _sc_gather(emb_pad, idx.reshape(bs * n))
    out = _post_residual(x_r, quantized.reshape(bs, n, 128), pW,
                         pb.reshape(c, 1))
    total_loss = (1.0 + 0.25) * jnp.sum(loss_parts) / (bs * d * h * w)
    return (out.reshape(bs, c, h, w), total_loss)


# speedup vs baseline: 1.1188x; 1.1188x over previous
import functools

import jax
import jax.numpy as jnp
from jax.experimental import pallas as pl
from jax.experimental.pallas import tpu as pltpu
from jax.experimental.pallas import tpu_sc as plsc


_CHUNK = 512
_GATHER_WINDOW = 256


def _distance_argmin(x_r, qW, qb2, emb, *, interpret=False):
    bs, c, n = x_r.shape
    k, d = emb.shape
    nchunks = k // _CHUNK

    nb = 2
    ngrid = bs // nb
    m = nb * n

    def body(x_ref, qw_ref, qb_ref, emb_ref, idx_ref, loss_ref, epad_ref):
        b = pl.program_id(0)
        echunk = emb_ref[pl.ds(b * (k // ngrid), k // ngrid), :]
        epad_ref[...] = jnp.concatenate(
            [echunk, jnp.zeros((k // ngrid, 128 - d), jnp.float32)], axis=1)
        qis = []
        for j in range(nb):
            xb = x_ref[j]
            qis.append(jax.lax.dot_general(
                qw_ref[...].astype(jnp.bfloat16), xb.astype(jnp.bfloat16),
                (((1,), (0,)), ((), ())), preferred_element_type=jnp.float32))
        qi = jnp.concatenate(qis, axis=1)
        qi = qi + qb_ref[...]
        q_sq = jnp.sum(qi * qi, axis=0, keepdims=True)
        qi_b = qi.astype(jnp.bfloat16)
        run_min = jnp.full((1, m), jnp.inf, jnp.float32)
        run_idx = jnp.zeros((1, m), jnp.int32)
        rows_f = jax.lax.broadcasted_iota(
            jnp.int32, (_CHUNK, m), 0).astype(jnp.float32)
        for kk in range(nchunks):
            eb = emb_ref[pl.ds(kk * _CHUNK, _CHUNK), :]
            esq = jnp.sum(eb * eb, axis=1, keepdims=True)
            dot2 = jax.lax.dot_general(
                (eb + eb).astype(jnp.bfloat16), qi_b,
                (((1,), (0,)), ((), ())), preferred_element_type=jnp.float32)
            s = (q_sq + esq) - dot2
            cmin = jnp.min(s, axis=0, keepdims=True)
            cidx_f = jnp.min(jnp.where(s == cmin, rows_f, float(k)),
                             axis=0, keepdims=True)
            cidx = cidx_f.astype(jnp.int32) + (kk * _CHUNK)
            better = cmin < run_min
            run_idx = jnp.where(better, cidx, run_idx)
            run_min = jnp.where(better, cmin, run_min)
        idx_ref[...] = run_idx
        loss_ref[0] = jnp.sum(jnp.maximum(run_min, 0.0), axis=1, keepdims=True)

    return pl.pallas_call(
        body,
        grid=(ngrid,),
        in_specs=[
            pl.BlockSpec((nb, c, n), lambda b: (b, 0, 0)),
            pl.BlockSpec((emb.shape[1], c), lambda b: (0, 0)),
            pl.BlockSpec((emb.shape[1], 1), lambda b: (0, 0)),
            pl.BlockSpec((k, d), lambda b: (0, 0)),
        ],
        out_specs=[
            pl.BlockSpec((1, m), lambda b: (0, b)),
            pl.BlockSpec((1, 1, 1), lambda b: (b, 0, 0)),
            pl.BlockSpec((k // ngrid, 128), lambda b: (b, 0)),
        ],
        out_shape=[
            jax.ShapeDtypeStruct((1, bs * n), jnp.int32),
            jax.ShapeDtypeStruct((ngrid, 1, 1), jnp.float32),
            jax.ShapeDtypeStruct((k, 128), jnp.float32),
        ],
        interpret=interpret,
    )(x_r, qW, qb2, emb)


def _sc_gather(emb, indices):
    total = indices.shape[0]
    k, d = emb.shape
    num_cores, num_subcores = 2, 16
    nw = num_cores * num_subcores
    b_per_w = total // nw
    mesh = plsc.VectorSubcoreMesh(core_axis_name="c", subcore_axis_name="s")

    @functools.partial(
        pl.kernel, mesh=mesh,
        out_type=jax.ShapeDtypeStruct((total, d), emb.dtype),
        scratch_types=[
            pltpu.VMEM((b_per_w,), jnp.int32),
            pltpu.VMEM((b_per_w, d), emb.dtype),
            pltpu.SemaphoreType.DMA,
        ],
    )
    def gather_kernel(emb_hbm, i_hbm, o_hbm, idx_v, rows_v, sem):
        wid = jax.lax.axis_index("s") * num_cores + jax.lax.axis_index("c")
        base = wid * b_per_w
        pltpu.sync_copy(i_hbm.at[pl.ds(base, b_per_w)], idx_v)
        pltpu.async_copy(emb_hbm.at[idx_v], rows_v, sem).wait()
        pltpu.sync_copy(rows_v, o_hbm.at[pl.ds(base, b_per_w)])

    return gather_kernel(emb, indices)


def _post_residual(x_r, q, pW, pb2, *, interpret=False):
    bs, c, n = x_r.shape
    d = pW.shape[1]
    qd = q.shape[-1]

    def body(x_ref, q_ref, pw_ref, pb_ref, out_ref):
        xb = x_ref[0]
        qv = q_ref[0]
        pw = jnp.concatenate(
            [pw_ref[...], jnp.zeros((c, qd - d), jnp.float32)], axis=1)
        post = jax.lax.dot_general(
            pw.astype(jnp.bfloat16), qv.astype(jnp.bfloat16),
            (((1,), (1,)), ((), ())), preferred_element_type=jnp.float32)
        out_ref[0] = xb - (post + pb_ref[...])

    return pl.pallas_call(
        body,
        grid=(bs,),
        in_specs=[
            pl.BlockSpec((1, c, n), lambda b: (b, 0, 0)),
            pl.BlockSpec((1, n, qd), lambda b: (b, 0, 0)),
            pl.BlockSpec((c, d), lambda b: (0, 0)),
            pl.BlockSpec((c, 1), lambda b: (0, 0)),
        ],
        out_specs=pl.BlockSpec((1, c, n), lambda b: (b, 0, 0)),
        out_shape=jax.ShapeDtypeStruct((bs, c, n), jnp.float32),
        interpret=interpret,
    )(x_r, q, pW, pb2)


def kernel(x, qW, qb, emb, pW, pb):
    bs, c, h, w = x.shape
    n = h * w
    d = emb.shape[1]
    x_r = x.reshape(bs, c, n)
    idx, loss_parts, emb_pad = _distance_argmin(x_r, qW, qb.reshape(d, 1), emb)
    quantized = _sc_gather(emb_pad, idx.reshape(bs * n))
    out = _post_residual(x_r, quantized.reshape(bs, n, 128), pW,
                         pb.reshape(c, 1))
    total_loss = (1.0 + 0.25) * jnp.sum(loss_parts) / (bs * d * h * w)
    return (out.reshape(bs, c, h, w), total_loss)
